# baseline (speedup 1.0000x reference)
import jax
import jax.numpy as jnp
from jax import lax
from jax.experimental import pallas as pl
from jax.experimental.pallas import tpu as pltpu
from jax.experimental.pallas import tpu_sc as plsc

_K = 8192
_D = 256
_RB = 8
_NSTEP = _K // 128 // _RB
_NT = 16
_CH = _K // _NT
_CW = _K // _NT


def _dist_body(z_ref, cb_ref, out_ref):
    z = z_ref[:].reshape(1, 1, _D)
    e = cb_ref[:] - z
    out_ref[:] = jnp.sum(e * e, axis=-1)


def _merge16(ak, av, bk, bv):
    rk = lax.rev(bk, (0,))
    rv = lax.rev(bv, (0,))
    sel = ak <= rk
    mk = jnp.where(sel, ak, rk)
    mv = jnp.where(sel, av, rv)
    return plsc.sort_key_val(mk, mv)


def _sc_body(dists_hbm, adj_hbm, sym_hbm, out_hbm,
             dch_v, tmpk_v, tmpv_v, rows_v, sym_v, candi_v, candd_v,
             partv_v, mgk_v, mgv_v, shacc_v, out_v,
             shk_s, shv_s, shc_s, shd_s, shp_s, sem):
    wid = lax.axis_index("s")
    base = wid * _CH
    pltpu.sync_copy(dists_hbm.at[pl.ds(base, _CH)], dch_v)
    pltpu.sync_copy(sym_hbm, sym_v)
    iota = lax.iota(jnp.int32, 16)

    bk = jnp.full((16,), jnp.inf, jnp.float32)
    bv = jnp.zeros((16,), jnp.int32)
    for c in range(_CH // 16):
        k = dch_v[pl.ds(c * 16, 16)]
        v = iota + (base + c * 16)
        ks, vs = plsc.sort_key_val(k, v)
        bk, bv = _merge16(ks, vs, bk, bv)
    tmpk_v[...] = bk
    tmpv_v[...] = bv
    pltpu.sync_copy(tmpk_v, shk_s.at[pl.ds(wid * 16, 16)])
    pltpu.sync_copy(tmpv_v, shv_s.at[pl.ds(wid * 16, 16)])
    plsc.subcore_barrier()

    @pl.when(wid == 0)
    def _():
        pltpu.sync_copy(shk_s, mgk_v)
        pltpu.sync_copy(shv_s, mgv_v)
        gk = mgk_v[pl.ds(0, 16)]
        gv = mgv_v[pl.ds(0, 16)]
        for t in range(1, _NT):
            gk, gv = _merge16(gk, gv,
                              mgk_v[pl.ds(t * 16, 16)],
                              mgv_v[pl.ds(t * 16, 16)])
        tmpk_v[...] = gk
        tmpv_v[...] = gv
        pltpu.sync_copy(tmpv_v, shc_s)
        pltpu.sync_copy(tmpk_v, shd_s)
    plsc.subcore_barrier()

    pltpu.sync_copy(shc_s, candi_v)
    pltpu.sync_copy(shd_s, candd_v)
    civ = candi_v[...]
    cdv = candd_v[...]
    cur = sym_v[...][0]
    colbase = wid * _CW
    cps = [pltpu.async_copy(adj_hbm.at[cur, pl.ds(colbase, _CW)],
                            rows_v.at[pl.ds(0, _CW)], sem)]
    for j in range(5):
        rj = civ[j]
        cps.append(pltpu.async_copy(adj_hbm.at[rj, pl.ds(colbase, _CW)],
                                    rows_v.at[pl.ds((j + 1) * _CW, _CW)], sem))
    for cp in cps:
        cp.wait()
    accs = [jnp.zeros((16,), jnp.float32) for _ in range(5)]
    for b in range(_CW // 16):
        r0 = rows_v[pl.ds(b * 16, 16)]
        for j in range(5):
            rj = rows_v[pl.ds((j + 1) * _CW + b * 16, 16)]
            accs[j] = accs[j] + jnp.abs(r0 - rj)
    for j in range(5):
        partv_v[...] = accs[j]
        pltpu.sync_copy(partv_v, shp_s.at[pl.ds((j * _NT + wid) * 16, 16)])
    plsc.subcore_barrier()

    @pl.when(wid == 0)
    def _():
        pltpu.sync_copy(shp_s, shacc_v)
        bs = jnp.float32(-jnp.inf)
        bi = jnp.int32(0)
        for j in range(5):
            a = jnp.zeros((16,), jnp.float32)
            for t in range(_NT):
                a = a + shacc_v[pl.ds((j * _NT + t) * 16, 16)]
            gd = jnp.sum(a) * jnp.float32(1.0 / _K)
            dj = cdv[j]
            ij = civ[j]
            s = -dj + jnp.float32(0.1) * gd
            s = jnp.where(ij == cur, -jnp.inf, s)
            if j == 0:
                bs, bi = s, ij
            else:
                take = s > bs
                bi = jnp.where(take, ij, bi)
                bs = jnp.maximum(bs, s)
        out_v[...] = jnp.full((16,), bi, jnp.int32)
        pltpu.sync_copy(out_v, out_hbm)


def kernel(z_flat, codebook, adjacency, current_sym):
    sym = jnp.broadcast_to(jnp.asarray(current_sym, dtype=jnp.int32), (16,))
    z2 = z_flat.reshape(1, _D)
    cb3 = codebook.reshape(_K // 128, 128, _D)
    dists = pl.pallas_call(
        _dist_body,
        grid=(_NSTEP,),
        in_specs=[
            pl.BlockSpec((1, _D), lambda i: (0, 0)),
            pl.BlockSpec((_RB, 128, _D), lambda i: (i, 0, 0)),
        ],
        out_specs=pl.BlockSpec((_RB, 128), lambda i: (i, 0)),
        out_shape=jax.ShapeDtypeStruct((_K // 128, 128), jnp.float32),
    )(z2, cb3)
    dflat = dists.reshape(_K)
    sc_fin = pl.kernel(
        _sc_body,
        out_type=jax.ShapeDtypeStruct((16,), jnp.int32),
        mesh=plsc.VectorSubcoreMesh(core_axis_name="c", subcore_axis_name="s",
                                    num_cores=1),
        compiler_params=pltpu.CompilerParams(needs_layout_passes=False),
        scratch_types=[
            pltpu.VMEM((_CH,), jnp.float32),
            pltpu.VMEM((16,), jnp.float32),
            pltpu.VMEM((16,), jnp.int32),
            pltpu.VMEM((6 * _CW,), jnp.float32),
            pltpu.VMEM((16,), jnp.int32),
            pltpu.VMEM((16,), jnp.int32),
            pltpu.VMEM((16,), jnp.float32),
            pltpu.VMEM((16,), jnp.float32),
            pltpu.VMEM((16 * _NT,), jnp.float32),
            pltpu.VMEM((16 * _NT,), jnp.int32),
            pltpu.VMEM((5 * _NT * 16,), jnp.float32),
            pltpu.VMEM((16,), jnp.int32),
            pltpu.VMEM_SHARED((16 * _NT,), jnp.float32),
            pltpu.VMEM_SHARED((16 * _NT,), jnp.int32),
            pltpu.VMEM_SHARED((16,), jnp.int32),
            pltpu.VMEM_SHARED((16,), jnp.float32),
            pltpu.VMEM_SHARED((5 * _NT * 16,), jnp.float32),
            pltpu.SemaphoreType.DMA,
        ],
    )
    out = sc_fin(dflat, adjacency, sym)
    return out[0]


# speedup vs baseline: 2.6268x; 2.6268x over previous
import jax
import jax.numpy as jnp
from jax import lax
from jax.experimental import pallas as pl
from jax.experimental.pallas import tpu as pltpu

_K = 8192
_D = 256
_NQ = 4
_RB = 2
_G = _K // 128
_QG = _G // _NQ
_NSTEP = _QG // _RB


def _body(sym_ref, z_ref, cb0, cb1, cb2, cb3, adj_ref, out_ref,
          dists_ref, rows_ref, sem):
    i = pl.program_id(0)

    @pl.when(i == 0)
    def _start_cur_row():
        pltpu.make_async_copy(
            adj_ref.at[pl.ds(sym_ref[0], 1)], rows_ref.at[pl.ds(0, 1)], sem
        ).start()

    @pl.when(i < _NSTEP)
    def _dist_step():
        z = z_ref[:].reshape(1, 1, _D)
        for q, cb in enumerate((cb0, cb1, cb2, cb3)):
            e = cb[:] - z
            d = jnp.sum(e * e, axis=-1)
            dists_ref[pl.ds(q * _QG + i * _RB, _RB), :] = d

    @pl.when(i == _NSTEP)
    def _select():
        d = dists_ref[:]
        ri = lax.broadcasted_iota(jnp.int32, d.shape, 0)
        ci = lax.broadcasted_iota(jnp.int32, d.shape, 1)
        flat = ri * 128 + ci
        cur = sym_ref[0]

        idxs, vals = [], []
        for t in range(5):
            m = jnp.min(d)
            idx = jnp.min(jnp.where(d == m, flat, jnp.int32(1 << 30)))
            pltpu.make_async_copy(
                adj_ref.at[pl.ds(idx, 1)], rows_ref.at[pl.ds(t + 1, 1)], sem
            ).start()
            idxs.append(idx)
            vals.append(m)
            d = jnp.where(flat == idx, jnp.float32(jnp.inf), d)

        for t in range(6):
            pltpu.make_async_copy(
                adj_ref.at[pl.ds(0, 1)], rows_ref.at[pl.ds(t, 1)], sem
            ).wait()

        cur_row = rows_ref[pl.ds(0, 1), :]
        best = jnp.int32(0)
        bs = jnp.float32(0)
        for t in range(5):
            gd = jnp.mean(jnp.abs(cur_row - rows_ref[pl.ds(t + 1, 1), :]))
            sc = -vals[t] + jnp.float32(0.1) * gd
            sc = jnp.where(idxs[t] == cur, -jnp.inf, sc)
            if t == 0:
                best, bs = idxs[t], sc
            else:
                take = sc > bs
                best = jnp.where(take, idxs[t], best)
                bs = jnp.maximum(bs, sc)
        out_ref[0] = best


def kernel(z_flat, codebook, adjacency, current_sym):
    sym = jnp.asarray(current_sym, dtype=jnp.int32).reshape(1)
    z2 = z_flat.reshape(1, _D)
    cb3 = codebook.reshape(_G, 128, _D)

    def _mk_spec(q):
        return pl.BlockSpec(
            (_RB, 128, _D),
            lambda i, q=q: (q * _NSTEP + jnp.minimum(i, _NSTEP - 1), 0, 0),
        )

    out = pl.pallas_call(
        _body,
        grid=(_NSTEP + 1,),
        in_specs=[
            pl.BlockSpec(memory_space=pltpu.SMEM),
            pl.BlockSpec((1, _D), lambda i: (0, 0)),
            _mk_spec(0), _mk_spec(1), _mk_spec(2), _mk_spec(3),
            pl.BlockSpec(memory_space=pl.ANY),
        ],
        out_specs=pl.BlockSpec(memory_space=pltpu.SMEM),
        out_shape=jax.ShapeDtypeStruct((1,), jnp.int32),
        scratch_shapes=[
            pltpu.VMEM((_G, 128), jnp.float32),
            pltpu.VMEM((8, _K), jnp.float32),
            pltpu.SemaphoreType.DMA,
        ],
    )(sym, z2, cb3, cb3, cb3, cb3, adjacency)
    return out[0]
